# Initial kernel scaffold; baseline (speedup 1.0000x reference)
import jax
import jax.numpy as jnp
from jax.experimental import pallas as pl


def kernel(x, edge_index, Wl1, Wr1, b1, Wl2, Wr2, b2, Wl3, Wr3, b3):
    raise NotImplementedError("write your pallas kernel here")


# speedup vs baseline: 3.7969x; 3.7969x over previous
import functools

import jax
import jax.numpy as jnp
from jax import lax
from jax.experimental import pallas as pl
from jax.experimental.pallas import tpu as pltpu
from jax.experimental.pallas import tpu_sc as plsc

N = 10000
E = 320000
NSC = 2
NTILES = 16
NP = 10240
HALF = NP // 2
RPA = HALF // NTILES
ACC_ROWS = HALF + 8
W = 128
EPT = E // NTILES
SUB = EPT // 2
CHUNK = 80
NCHS = SUB // CHUNK


def _sc_mesh():
    return plsc.VectorSubcoreMesh(core_axis_name="c", subcore_axis_name="s")


def _make_agg(splits: int, count: bool):
    rounds = splits

    if splits == 1:
        out_type = [jax.ShapeDtypeStruct((NP, W), jnp.float32)]
    else:
        out_type = [jax.ShapeDtypeStruct((NP, W), jnp.float32),
                    jax.ShapeDtypeStruct((NP, W), jnp.float32)]
    if count:
        out_type.append(jax.ShapeDtypeStruct((NP,), jnp.float32))

    scratch = [
        pltpu.VMEM((SUB,), jnp.int32),
        pltpu.VMEM((SUB,), jnp.int32),
        pltpu.VMEM((CHUNK,), jnp.int32),
        pltpu.VMEM((CHUNK,), jnp.int32),
        pltpu.VMEM((CHUNK, W), jnp.float32),
        pltpu.VMEM_SHARED((ACC_ROWS, W), jnp.float32),
        pltpu.SemaphoreType.DMA,
    ]
    if count:
        scratch.append(pltpu.VMEM((CHUNK,), jnp.float32))
        scratch.append(pltpu.VMEM((RPA,), jnp.float32))
        scratch.append(pltpu.VMEM_SHARED((ACC_ROWS,), jnp.float32))

    @functools.partial(pl.kernel, out_type=out_type, mesh=_sc_mesh(),
                       scratch_types=scratch,
                       name=f"sage_agg_s{splits}" + ("_cnt" if count else ""))
    def k(*args):
        y, srci, dsti, zrows = args[:4]
        args = args[4:]
        if count:
            zn, ones_in = args[:2]
            args = args[2:]
        outs = args[:splits]
        args = args[splits:]
        if count:
            cnto = args[0]
            args = args[1:]
        sall, dall, gidx, lidx, rows, acc, sem = args[:7]
        if count:
            ones, cbuf, cacc = args[7:]

        c = lax.axis_index("c")
        s = lax.axis_index("s")
        zsl = pl.ds(s * RPA, RPA)

        if count:
            pltpu.sync_copy(ones_in, ones)
            pltpu.sync_copy(zn.at[zsl], cbuf)
            pltpu.sync_copy(cbuf, cacc.at[zsl])

        for r in range(rounds):
            rowbase = (r if splits > 1 else c) * HALF
            pltpu.sync_copy(zrows.at[zsl], acc.at[zsl])
            plsc.subcore_barrier()

            for h in range(2):
                ebase = s * EPT + h * SUB
                pltpu.sync_copy(srci.at[pl.ds(ebase, SUB)], sall)
                pltpu.sync_copy(dsti.at[pl.ds(ebase, SUB)], dall)

                def body(j, carry):
                    o = j * CHUNK
                    for i in range(CHUNK // 16):
                        st = pl.ds(i * 16, 16)
                        dyn = pl.ds(o + i * 16, 16)
                        if splits == 1:
                            gidx[st] = sall[dyn]
                        else:
                            gidx[st] = sall[dyn] * 2 + c
                        l = dall[dyn] - rowbase
                        ok = jnp.logical_and(l >= 0, l < HALF)
                        lidx[st] = jnp.where(ok, l, HALF)
                    pltpu.async_copy(y.at[gidx], rows, sem).wait()
                    pltpu.sync_copy(rows, acc.at[lidx], add=True)
                    if count:
                        pltpu.sync_copy(ones, cacc.at[lidx], add=True)
                    return carry

                lax.fori_loop(0, NCHS, body, 0)

            plsc.subcore_barrier()

            if splits == 1:
                osl = pl.ds(c * HALF + s * RPA, RPA)
                pltpu.sync_copy(acc.at[zsl], outs[0].at[osl])
            else:
                osl = pl.ds(r * HALF + s * RPA, RPA)

                @pl.when(c == 0)
                def _():
                    pltpu.sync_copy(acc.at[zsl], outs[0].at[osl])

                @pl.when(c == 1)
                def _():
                    pltpu.sync_copy(acc.at[zsl], outs[1].at[osl])

            if r + 1 < rounds:
                plsc.subcore_barrier()

        if count:
            pltpu.sync_copy(cacc.at[zsl], cbuf)
            pltpu.sync_copy(cbuf, cnto.at[pl.ds(c * HALF + s * RPA, RPA)])

    return k


_agg_l1 = _make_agg(splits=1, count=True)
_agg_l3 = _make_agg(splits=1, count=False)
_agg_l2 = _make_agg(splits=2, count=False)


_BR = 640


def _row_spec(w):
    return pl.BlockSpec((_BR, w), lambda i: (i, 0))


def _full_spec(a, b):
    return pl.BlockSpec((a, b), lambda i: (0, 0))


def _tc_l1(p, cnt, x, Wl1, Wr1, b1):
    def body(pr, cr, xr, wl, wr, br, out):
        inv = 1.0 / jnp.maximum(cr[...], 1.0)
        z = (jnp.dot(pr[...] * inv, wl[...], preferred_element_type=jnp.float32)
             + jnp.dot(xr[...], wr[...], preferred_element_type=jnp.float32)
             + br[...])
        out[...] = jnp.maximum(z, 0.0)

    return pl.pallas_call(
        body,
        grid=(NP // _BR,),
        in_specs=[_row_spec(128), _row_spec(1), _row_spec(128),
                  _full_spec(128, 256), _full_spec(128, 256),
                  _full_spec(1, 256)],
        out_specs=_row_spec(256),
        out_shape=jax.ShapeDtypeStruct((NP, 256), jnp.float32),
    )(p, cnt, x, Wl1, Wr1, b1)


def _tc_l2(a0, a1, cnt, h1, Wl2a, Wl2b, Wr2, b2, Wl3):
    def body(a0r, a1r, cr, hr, wla, wlb, wr, br, wl3, h2o, y3o):
        inv = 1.0 / jnp.maximum(cr[...], 1.0)
        z = (jnp.dot(a0r[...] * inv, wla[...], preferred_element_type=jnp.float32)
             + jnp.dot(a1r[...] * inv, wlb[...], preferred_element_type=jnp.float32)
             + jnp.dot(hr[...], wr[...], preferred_element_type=jnp.float32)
             + br[...])
        h2 = jnp.maximum(z, 0.0)
        h2o[...] = h2
        y3o[...] = jnp.dot(h2, wl3[...], preferred_element_type=jnp.float32)

    return pl.pallas_call(
        body,
        grid=(NP // _BR,),
        in_specs=[_row_spec(128), _row_spec(128), _row_spec(1),
                  _row_spec(256), _full_spec(128, 256), _full_spec(128, 256),
                  _full_spec(256, 256), _full_spec(1, 256),
                  _full_spec(256, 128)],
        out_specs=[_row_spec(256), _row_spec(128)],
        out_shape=[jax.ShapeDtypeStruct((NP, 256), jnp.float32),
                   jax.ShapeDtypeStruct((NP, 128), jnp.float32)],
    )(a0, a1, cnt, h1, Wl2a, Wl2b, Wr2, b2, Wl3)


def _tc_l3(q, cnt, h2, Wr3, b3):
    def body(qr, cr, hr, wr, br, out):
        inv = 1.0 / jnp.maximum(cr[...], 1.0)
        out[...] = (qr[...] * inv
                    + jnp.dot(hr[...], wr[...],
                              preferred_element_type=jnp.float32)
                    + br[...])

    return pl.pallas_call(
        body,
        grid=(NP // _BR,),
        in_specs=[_row_spec(128), _row_spec(1), _row_spec(256),
                  _full_spec(256, 128), _full_spec(1, 128)],
        out_specs=_row_spec(128),
        out_shape=jax.ShapeDtypeStruct((NP, 128), jnp.float32),
    )(q, cnt, h2, Wr3, b3)


def kernel(x, edge_index, Wl1, Wr1, b1, Wl2, Wr2, b2, Wl3, Wr3, b3):
    src = edge_index[0].astype(jnp.int32)
    dst = edge_index[1].astype(jnp.int32)
    zrows = jnp.zeros((NP, W), jnp.float32)
    zn = jnp.zeros((NP,), jnp.float32)
    ones = jnp.ones((CHUNK,), jnp.float32)

    xp = jnp.zeros((NP, 128), jnp.float32).at[:N].set(x)
    p, cnt = _agg_l1(xp, src, dst, zrows, zn, ones)
    cnt = cnt.reshape(NP, 1)
    h1 = _tc_l1(p, cnt, xp, Wl1, Wr1, b1.reshape(1, 256))

    a0, a1 = _agg_l2(h1.reshape(2 * NP, W), src, dst, zrows)
    h2, y3 = _tc_l2(a0, a1, cnt, h1, Wl2[:W], Wl2[W:], Wr2,
                    b2.reshape(1, 256), Wl3)

    q = _agg_l3(y3, src, dst, zrows)
    out = _tc_l3(q[0], cnt, h2, Wr3, b3.reshape(1, 128))
    return out[:N]


# speedup vs baseline: 6.8742x; 1.8105x over previous
import functools

import jax
import jax.numpy as jnp
from jax import lax
from jax.experimental import pallas as pl
from jax.experimental.pallas import tpu as pltpu
from jax.experimental.pallas import tpu_sc as plsc

N = 10000
E = 320000
NSC = 2
NTILES = 16
NP = 10240
RPZ = NP // NTILES
W = 128
SUB = 10000
CHUNK = 80
NCHS = SUB // CHUNK


def _sc_mesh():
    return plsc.VectorSubcoreMesh(core_axis_name="c", subcore_axis_name="s")


def _make_agg(splits: int, count: bool):
    stages = 2 if splits == 2 else 1

    out_type = [jax.ShapeDtypeStruct((NP, W), jnp.float32),
                jax.ShapeDtypeStruct((NP, W), jnp.float32)]
    if count:
        out_type.append(jax.ShapeDtypeStruct((2 * NP,), jnp.float32))

    scratch = [
        pltpu.VMEM((SUB,), jnp.int32),
        pltpu.VMEM((SUB,), jnp.int32),
        pltpu.VMEM((CHUNK,), jnp.int32),
        pltpu.VMEM((CHUNK,), jnp.int32),
        pltpu.VMEM((CHUNK, W), jnp.float32),
        pltpu.VMEM_SHARED((NP, W), jnp.float32),
        pltpu.SemaphoreType.DMA,
    ]
    if count:
        scratch.append(pltpu.VMEM((CHUNK,), jnp.float32))
        scratch.append(pltpu.VMEM((RPZ,), jnp.float32))
        scratch.append(pltpu.VMEM_SHARED((NP,), jnp.float32))

    @functools.partial(pl.kernel, out_type=out_type, mesh=_sc_mesh(),
                       scratch_types=scratch,
                       name=f"sage_agg_s{splits}" + ("_cnt" if count else ""))
    def k(*args):
        y, srci, dsti, zrows = args[:4]
        args = args[4:]
        if count:
            zn, ones_in = args[:2]
            args = args[2:]
        outs = args[:2]
        args = args[2:]
        if count:
            cnto = args[0]
            args = args[1:]
        sall, dall, gidx, lidx, rows, acc, sem = args[:7]
        if count:
            ones, cbuf, cacc = args[7:]

        c = lax.axis_index("c")
        s = lax.axis_index("s")
        zsl = pl.ds(s * RPZ, RPZ)

        if count:
            pltpu.sync_copy(ones_in, ones)
            pltpu.sync_copy(zn.at[zsl], cbuf)
            pltpu.sync_copy(cbuf, cacc.at[zsl])

        pltpu.sync_copy(zrows.at[zsl], acc.at[zsl])
        plsc.subcore_barrier()

        for h in range(stages):
            if splits == 1:
                ebase = c * (E // 2) + s * SUB
            else:
                ebase = s * (stages * SUB) + h * SUB
            pltpu.sync_copy(srci.at[pl.ds(ebase, SUB)], sall)
            pltpu.sync_copy(dsti.at[pl.ds(ebase, SUB)], dall)

            def body(j, carry):
                o = j * CHUNK
                for i in range(CHUNK // 16):
                    st = pl.ds(i * 16, 16)
                    dyn = pl.ds(o + i * 16, 16)
                    if splits == 1:
                        gidx[st] = sall[dyn]
                    else:
                        gidx[st] = sall[dyn] * 2 + c
                    lidx[st] = dall[dyn]
                pltpu.async_copy(y.at[gidx], rows, sem).wait()
                pltpu.sync_copy(rows, acc.at[lidx], add=True)
                if count:
                    pltpu.sync_copy(ones, cacc.at[lidx], add=True)
                return carry

            lax.fori_loop(0, NCHS, body, 0)

        plsc.subcore_barrier()

        @pl.when(c == 0)
        def _():
            pltpu.sync_copy(acc.at[zsl], outs[0].at[zsl])

        @pl.when(c == 1)
        def _():
            pltpu.sync_copy(acc.at[zsl], outs[1].at[zsl])

        if count:
            pltpu.sync_copy(cacc.at[zsl], cbuf)
            pltpu.sync_copy(cbuf, cnto.at[pl.ds(c * NP + s * RPZ, RPZ)])

    return k


_agg_l1 = _make_agg(splits=1, count=True)
_agg_l3 = _make_agg(splits=1, count=False)
_agg_l2 = _make_agg(splits=2, count=False)


_BR = 640


def _row_spec(w):
    return pl.BlockSpec((_BR, w), lambda i: (i, 0))


def _full_spec(a, b):
    return pl.BlockSpec((a, b), lambda i: (0, 0))


def _tc_l1(p0, p1, cnt0, cnt1, x, Wl1, Wr1, b1):
    def body(p0r, p1r, c0r, c1r, xr, wl, wr, br, out, cnto):
        cnt = c0r[...] + c1r[...]
        inv = 1.0 / jnp.maximum(cnt, 1.0)
        z = (jnp.dot((p0r[...] + p1r[...]) * inv, wl[...],
                     preferred_element_type=jnp.float32)
             + jnp.dot(xr[...], wr[...], preferred_element_type=jnp.float32)
             + br[...])
        out[...] = jnp.maximum(z, 0.0)
        cnto[...] = cnt

    return pl.pallas_call(
        body,
        grid=(NP // _BR,),
        in_specs=[_row_spec(128), _row_spec(128), _row_spec(1), _row_spec(1),
                  _row_spec(128), _full_spec(128, 256), _full_spec(128, 256),
                  _full_spec(1, 256)],
        out_specs=[_row_spec(256), _row_spec(1)],
        out_shape=[jax.ShapeDtypeStruct((NP, 256), jnp.float32),
                   jax.ShapeDtypeStruct((NP, 1), jnp.float32)],
    )(p0, p1, cnt0, cnt1, x, Wl1, Wr1, b1)


def _tc_l2(a0, a1, cnt, h1, Wl2a, Wl2b, Wr2, b2, Wl3):
    def body(a0r, a1r, cr, hr, wla, wlb, wr, br, wl3, h2o, y3o):
        inv = 1.0 / jnp.maximum(cr[...], 1.0)
        z = (jnp.dot(a0r[...] * inv, wla[...], preferred_element_type=jnp.float32)
             + jnp.dot(a1r[...] * inv, wlb[...], preferred_element_type=jnp.float32)
             + jnp.dot(hr[...], wr[...], preferred_element_type=jnp.float32)
             + br[...])
        h2 = jnp.maximum(z, 0.0)
        h2o[...] = h2
        y3o[...] = jnp.dot(h2, wl3[...], preferred_element_type=jnp.float32)

    return pl.pallas_call(
        body,
        grid=(NP // _BR,),
        in_specs=[_row_spec(128), _row_spec(128), _row_spec(1),
                  _row_spec(256), _full_spec(128, 256), _full_spec(128, 256),
                  _full_spec(256, 256), _full_spec(1, 256),
                  _full_spec(256, 128)],
        out_specs=[_row_spec(256), _row_spec(128)],
        out_shape=[jax.ShapeDtypeStruct((NP, 256), jnp.float32),
                   jax.ShapeDtypeStruct((NP, 128), jnp.float32)],
    )(a0, a1, cnt, h1, Wl2a, Wl2b, Wr2, b2, Wl3)


def _tc_l3(q0, q1, cnt, h2, Wr3, b3):
    def body(q0r, q1r, cr, hr, wr, br, out):
        inv = 1.0 / jnp.maximum(cr[...], 1.0)
        out[...] = ((q0r[...] + q1r[...]) * inv
                    + jnp.dot(hr[...], wr[...],
                              preferred_element_type=jnp.float32)
                    + br[...])

    return pl.pallas_call(
        body,
        grid=(NP // _BR,),
        in_specs=[_row_spec(128), _row_spec(128), _row_spec(1),
                  _row_spec(256), _full_spec(256, 128), _full_spec(1, 128)],
        out_specs=_row_spec(128),
        out_shape=jax.ShapeDtypeStruct((NP, 128), jnp.float32),
    )(q0, q1, cnt, h2, Wr3, b3)


def kernel(x, edge_index, Wl1, Wr1, b1, Wl2, Wr2, b2, Wl3, Wr3, b3):
    src = edge_index[0].astype(jnp.int32)
    dst = edge_index[1].astype(jnp.int32)
    zrows = jnp.zeros((NP, W), jnp.float32)
    zn = jnp.zeros((NP,), jnp.float32)
    ones = jnp.ones((CHUNK,), jnp.float32)

    xp = jnp.zeros((NP, 128), jnp.float32).at[:N].set(x)
    p0, p1, cnt2 = _agg_l1(xp, src, dst, zrows, zn, ones)
    cnt2 = cnt2.reshape(2, NP, 1)
    h1, cnt = _tc_l1(p0, p1, cnt2[0], cnt2[1], xp, Wl1, Wr1,
                     b1.reshape(1, 256))

    a0, a1 = _agg_l2(h1.reshape(2 * NP, W), src, dst, zrows)
    h2, y3 = _tc_l2(a0, a1, cnt, h1, Wl2[:W], Wl2[W:], Wr2,
                    b2.reshape(1, 256), Wl3)

    q0, q1 = _agg_l3(y3, src, dst, zrows)
    out = _tc_l3(q0, q1, cnt, h2, Wr3, b3.reshape(1, 128))
    return out[:N]


# speedup vs baseline: 11.1671x; 1.6245x over previous
import functools

import jax
import jax.numpy as jnp
from jax import lax
from jax.experimental import pallas as pl
from jax.experimental.pallas import tpu as pltpu
from jax.experimental.pallas import tpu_sc as plsc

N = 10000
E = 320000
NSC = 2
NTILES = 16
NP = 10240
RPZ = NP // NTILES
W = 128
SUB = 10000
CHUNK = 80
NCHS = SUB // CHUNK


def _sc_mesh():
    return plsc.VectorSubcoreMesh(core_axis_name="c", subcore_axis_name="s")


def _make_agg(splits: int, count: bool):
    stages = 2 if splits == 2 else 1

    out_type = [jax.ShapeDtypeStruct((NP, W), jnp.float32),
                jax.ShapeDtypeStruct((NP, W), jnp.float32)]
    if count:
        out_type.append(jax.ShapeDtypeStruct((2 * NP,), jnp.float32))

    scratch = [
        pltpu.VMEM((SUB,), jnp.int32),
        pltpu.VMEM((SUB,), jnp.int32),
        pltpu.VMEM((CHUNK,), jnp.int32),
        pltpu.VMEM((CHUNK,), jnp.int32),
        pltpu.VMEM((CHUNK, W), jnp.float32),
        pltpu.VMEM((CHUNK,), jnp.int32),
        pltpu.VMEM((CHUNK,), jnp.int32),
        pltpu.VMEM((CHUNK, W), jnp.float32),
        pltpu.VMEM_SHARED((NP, W), jnp.float32),
        pltpu.SemaphoreType.DMA,
        pltpu.SemaphoreType.DMA,
    ]
    if count:
        scratch.append(pltpu.VMEM((CHUNK,), jnp.float32))
        scratch.append(pltpu.VMEM((RPZ,), jnp.float32))
        scratch.append(pltpu.VMEM_SHARED((NP,), jnp.float32))

    @functools.partial(pl.kernel, out_type=out_type, mesh=_sc_mesh(),
                       scratch_types=scratch,
                       name=f"sage_agg_s{splits}" + ("_cnt" if count else ""))
    def k(*args):
        y, srci, dsti, zrows = args[:4]
        args = args[4:]
        if count:
            zn, ones_in = args[:2]
            args = args[2:]
        outs = args[:2]
        args = args[2:]
        if count:
            cnto = args[0]
            args = args[1:]
        (sall, dall, gidxa, lidxa, rowsa, gidxb, lidxb, rowsb,
         acc, sema, semb) = args[:11]
        if count:
            ones, cbuf, cacc = args[11:]

        c = lax.axis_index("c")
        s = lax.axis_index("s")
        zsl = pl.ds(s * RPZ, RPZ)

        if count:
            pltpu.sync_copy(ones_in, ones)
            pltpu.sync_copy(zn.at[zsl], cbuf)
            pltpu.sync_copy(cbuf, cacc.at[zsl])

        pltpu.sync_copy(zrows.at[zsl], acc.at[zsl])
        plsc.subcore_barrier()

        def fill(gix, lix, o):
            for i in range(CHUNK // 16):
                st = pl.ds(i * 16, 16)
                dyn = pl.ds(o + i * 16, 16)
                if splits == 1:
                    gix[st] = sall[dyn]
                else:
                    gix[st] = sall[dyn] * 2 + c
                lix[st] = dall[dyn]

        def scat(rws, lix):
            pltpu.sync_copy(rws, acc.at[lix], add=True)
            if count:
                pltpu.sync_copy(ones, cacc.at[lix], add=True)

        for h in range(stages):
            if splits == 1:
                ebase = c * (E // 2) + s * SUB
            else:
                ebase = s * (stages * SUB) + h * SUB
            pltpu.sync_copy(srci.at[pl.ds(ebase, SUB)], sall)
            pltpu.sync_copy(dsti.at[pl.ds(ebase, SUB)], dall)

            fill(gidxa, lidxa, 0)
            pltpu.async_copy(y.at[gidxa], rowsa, sema)

            def body(t, carry):
                o = t * (2 * CHUNK)
                fill(gidxb, lidxb, o + CHUNK)
                pltpu.async_copy(y.at[gidxb], rowsb, semb)
                pltpu.make_async_copy(y.at[gidxa], rowsa, sema).wait()
                scat(rowsa, lidxa)
                fill(gidxa, lidxa, o + 2 * CHUNK)
                pltpu.async_copy(y.at[gidxa], rowsa, sema)
                pltpu.make_async_copy(y.at[gidxb], rowsb, semb).wait()
                scat(rowsb, lidxb)
                return carry

            lax.fori_loop(0, (NCHS - 1) // 2, body, 0)
            pltpu.make_async_copy(y.at[gidxa], rowsa, sema).wait()
            scat(rowsa, lidxa)

        plsc.subcore_barrier()

        @pl.when(c == 0)
        def _():
            pltpu.sync_copy(acc.at[zsl], outs[0].at[zsl])

        @pl.when(c == 1)
        def _():
            pltpu.sync_copy(acc.at[zsl], outs[1].at[zsl])

        if count:
            pltpu.sync_copy(cacc.at[zsl], cbuf)
            pltpu.sync_copy(cbuf, cnto.at[pl.ds(c * NP + s * RPZ, RPZ)])

    return k


_agg_l1 = _make_agg(splits=1, count=True)
_agg_l3 = _make_agg(splits=1, count=False)
_agg_l2 = _make_agg(splits=2, count=False)


_BR = 640


def _row_spec(w):
    return pl.BlockSpec((_BR, w), lambda i: (i, 0))


def _full_spec(a, b):
    return pl.BlockSpec((a, b), lambda i: (0, 0))


def _tc_l1(p0, p1, cnt0, cnt1, x, Wl1, Wr1, b1):
    def body(p0r, p1r, c0r, c1r, xr, wl, wr, br, out, cnto):
        cnt = c0r[...] + c1r[...]
        inv = 1.0 / jnp.maximum(cnt, 1.0)
        z = (jnp.dot((p0r[...] + p1r[...]) * inv, wl[...],
                     preferred_element_type=jnp.float32)
             + jnp.dot(xr[...], wr[...], preferred_element_type=jnp.float32)
             + br[...])
        out[...] = jnp.maximum(z, 0.0)
        cnto[...] = cnt

    return pl.pallas_call(
        body,
        grid=(NP // _BR,),
        in_specs=[_row_spec(128), _row_spec(128), _row_spec(1), _row_spec(1),
                  _row_spec(128), _full_spec(128, 256), _full_spec(128, 256),
                  _full_spec(1, 256)],
        out_specs=[_row_spec(256), _row_spec(1)],
        out_shape=[jax.ShapeDtypeStruct((NP, 256), jnp.float32),
                   jax.ShapeDtypeStruct((NP, 1), jnp.float32)],
    )(p0, p1, cnt0, cnt1, x, Wl1, Wr1, b1)


def _tc_l2(a0, a1, cnt, h1, Wl2a, Wl2b, Wr2, b2, Wl3):
    def body(a0r, a1r, cr, hr, wla, wlb, wr, br, wl3, h2o, y3o):
        inv = 1.0 / jnp.maximum(cr[...], 1.0)
        z = (jnp.dot(a0r[...] * inv, wla[...], preferred_element_type=jnp.float32)
             + jnp.dot(a1r[...] * inv, wlb[...], preferred_element_type=jnp.float32)
             + jnp.dot(hr[...], wr[...], preferred_element_type=jnp.float32)
             + br[...])
        h2 = jnp.maximum(z, 0.0)
        h2o[...] = h2
        y3o[...] = jnp.dot(h2, wl3[...], preferred_element_type=jnp.float32)

    return pl.pallas_call(
        body,
        grid=(NP // _BR,),
        in_specs=[_row_spec(128), _row_spec(128), _row_spec(1),
                  _row_spec(256), _full_spec(128, 256), _full_spec(128, 256),
                  _full_spec(256, 256), _full_spec(1, 256),
                  _full_spec(256, 128)],
        out_specs=[_row_spec(256), _row_spec(128)],
        out_shape=[jax.ShapeDtypeStruct((NP, 256), jnp.float32),
                   jax.ShapeDtypeStruct((NP, 128), jnp.float32)],
    )(a0, a1, cnt, h1, Wl2a, Wl2b, Wr2, b2, Wl3)


def _tc_l3(q0, q1, cnt, h2, Wr3, b3):
    def body(q0r, q1r, cr, hr, wr, br, out):
        inv = 1.0 / jnp.maximum(cr[...], 1.0)
        out[...] = ((q0r[...] + q1r[...]) * inv
                    + jnp.dot(hr[...], wr[...],
                              preferred_element_type=jnp.float32)
                    + br[...])

    return pl.pallas_call(
        body,
        grid=(NP // _BR,),
        in_specs=[_row_spec(128), _row_spec(128), _row_spec(1),
                  _row_spec(256), _full_spec(256, 128), _full_spec(1, 128)],
        out_specs=_row_spec(128),
        out_shape=jax.ShapeDtypeStruct((NP, 128), jnp.float32),
    )(q0, q1, cnt, h2, Wr3, b3)


def kernel(x, edge_index, Wl1, Wr1, b1, Wl2, Wr2, b2, Wl3, Wr3, b3):
    src = edge_index[0].astype(jnp.int32)
    dst = edge_index[1].astype(jnp.int32)
    zrows = jnp.zeros((NP, W), jnp.float32)
    zn = jnp.zeros((NP,), jnp.float32)
    ones = jnp.ones((CHUNK,), jnp.float32)

    xp = jnp.zeros((NP, 128), jnp.float32).at[:N].set(x)
    p0, p1, cnt2 = _agg_l1(xp, src, dst, zrows, zn, ones)
    cnt2 = cnt2.reshape(2, NP, 1)
    h1, cnt = _tc_l1(p0, p1, cnt2[0], cnt2[1], xp, Wl1, Wr1,
                     b1.reshape(1, 256))

    a0, a1 = _agg_l2(h1.reshape(2 * NP, W), src, dst, zrows)
    h2, y3 = _tc_l2(a0, a1, cnt, h1, Wl2[:W], Wl2[W:], Wr2,
                    b2.reshape(1, 256), Wl3)

    q0, q1 = _agg_l3(y3, src, dst, zrows)
    out = _tc_l3(q0, q1, cnt, h2, Wr3, b3.reshape(1, 128))
    return out[:N]
